# baseline (speedup 1.0000x reference)
import jax
import jax.numpy as jnp
from jax.experimental import pallas as pl
from jax.experimental.pallas import tpu as pltpu

S = 2048
D = 1024
E = 8
DFF = 4096
CAP = 512

_INTERPRET = False


def _gating_kernel(hs_ref, wg_ref, route_ref, laux_ref, cnt_ref):
    hs = hs_ref[...]
    wg = wg_ref[...]
    logits = jnp.dot(hs, wg, preferred_element_type=jnp.float32)
    gates = jax.nn.softmax(logits, axis=-1)
    iota_e = jax.lax.broadcasted_iota(jnp.int32, (S, E), 1)
    gmax = jnp.max(gates, axis=1, keepdims=True)
    idx1 = jnp.min(jnp.where(gates == gmax, iota_e, E), axis=1, keepdims=True)
    mask1 = iota_e == idx1
    lx = jnp.where(mask1, -jnp.inf, logits)
    lmax = jnp.max(lx, axis=1, keepdims=True)
    idx2 = jnp.min(jnp.where(lx == lmax, iota_e, E), axis=1, keepdims=True)
    mask2 = iota_e == idx2
    m1f = mask1.astype(jnp.float32)
    m2f = mask2.astype(jnp.float32)
    ir = jax.lax.broadcasted_iota(jnp.int32, (S, S), 0)
    ic = jax.lax.broadcasted_iota(jnp.int32, (S, S), 1)
    ltri = (ic < ir).astype(jnp.float32)
    loc1 = jnp.dot(ltri, m1f, preferred_element_type=jnp.float32)
    count1 = jnp.sum(m1f, axis=0, keepdims=True)
    loc2 = jnp.dot(ltri, m2f, preferred_element_type=jnp.float32) + count1
    me = jnp.mean(gates, axis=0, keepdims=True)
    ce = jnp.mean(m1f, axis=0, keepdims=True)
    laux_ref[...] = jnp.sum(me * ce, axis=1, keepdims=True) * jnp.float32(E)
    cnt_ref[...] = count1.astype(jnp.int32)
    c1 = jnp.sum(jnp.where(mask1, loc1, 0.0), axis=1, keepdims=True)
    c2 = jnp.sum(jnp.where(mask2, loc2, 0.0), axis=1, keepdims=True)
    k1 = c1 < CAP
    k2 = c2 < CAP
    g1 = jnp.where(k1, jnp.sum(jnp.where(mask1, gates, 0.0), axis=1, keepdims=True), 0.0)
    g2 = jnp.where(k2, jnp.sum(jnp.where(mask2, gates, 0.0), axis=1, keepdims=True), 0.0)
    denom = jnp.maximum(g1 + g2, jnp.finfo(jnp.float32).eps)
    g1 = g1 / denom
    g2 = g2 / denom
    route = jnp.concatenate(
        [idx1.astype(jnp.float32), idx2.astype(jnp.float32), c1, c2, g1, g2,
         jnp.zeros((S, 2), jnp.float32)], axis=1)
    route_ref[...] = route


def _expert_kernel(route_ref, hsb_ref, w1_ref, b1_ref, w2_ref, b2_ref, out_ref):
    e = pl.program_id(0)
    idx1 = route_ref[:, 0:1].astype(jnp.int32)
    idx2 = route_ref[:, 1:2].astype(jnp.int32)
    c1 = route_ref[:, 2:3].astype(jnp.int32)
    c2 = route_ref[:, 3:4].astype(jnp.int32)
    g1 = jnp.where(idx1 == e, route_ref[:, 4:5], 0.0)
    g2 = jnp.where(idx2 == e, route_ref[:, 5:6], 0.0)
    iota_c = jax.lax.broadcasted_iota(jnp.int32, (S, CAP), 1)
    cmb_f = (jnp.where(c1 == iota_c, g1, 0.0)
             + jnp.where(c2 == iota_c, g2, 0.0))
    cmb = cmb_f.astype(jnp.bfloat16)
    mask = jnp.where(cmb_f > 0.0, 1.0, 0.0).astype(jnp.bfloat16)
    dispatched = jax.lax.dot_general(
        mask, hsb_ref[...], (((0,), (0,)), ((), ())),
        preferred_element_type=jnp.float32).astype(jnp.bfloat16)
    h = jnp.maximum(
        jnp.dot(dispatched, w1_ref[0], preferred_element_type=jnp.float32)
        + b1_ref[0, 0][None, :], 0.0).astype(jnp.bfloat16)
    eo = (jnp.dot(h, w2_ref[0], preferred_element_type=jnp.float32)
          + b2_ref[0, 0][None, :]).astype(jnp.bfloat16)
    contrib = jnp.dot(cmb, eo, preferred_element_type=jnp.float32)

    @pl.when(e == 0)
    def _():
        out_ref[...] = contrib

    @pl.when(e != 0)
    def _():
        out_ref[...] += contrib


def kernel(hidden_states, wg, w1, b1, w2, b2):
    route, laux, counts = pl.pallas_call(
        _gating_kernel,
        out_shape=(
            jax.ShapeDtypeStruct((S, 8), jnp.float32),
            jax.ShapeDtypeStruct((1, 1), jnp.float32),
            jax.ShapeDtypeStruct((1, E), jnp.int32),
        ),
        interpret=_INTERPRET,
    )(hidden_states, wg)

    hsb = hidden_states.astype(jnp.bfloat16)
    w1b = w1.astype(jnp.bfloat16)
    w2b = w2.astype(jnp.bfloat16)

    out = pl.pallas_call(
        _expert_kernel,
        grid=(E,),
        in_specs=[
            pl.BlockSpec((S, 8), lambda e: (0, 0)),
            pl.BlockSpec((S, D), lambda e: (0, 0)),
            pl.BlockSpec((1, D, DFF), lambda e: (e, 0, 0)),
            pl.BlockSpec((1, 1, DFF), lambda e: (e, 0, 0)),
            pl.BlockSpec((1, DFF, D), lambda e: (e, 0, 0)),
            pl.BlockSpec((1, 1, D), lambda e: (e, 0, 0)),
        ],
        out_specs=pl.BlockSpec((S, D), lambda e: (0, 0)),
        out_shape=jax.ShapeDtypeStruct((S, D), jnp.float32),
        compiler_params=pltpu.CompilerParams(
            dimension_semantics=("arbitrary",),
        ),
        interpret=_INTERPRET,
    )(route, hsb, w1b, b1.reshape(E, 1, DFF), w2b, b2.reshape(E, 1, D))

    return out, laux.reshape(()), counts.reshape((E,))


# speedup vs baseline: 1.5058x; 1.5058x over previous
import jax
import jax.numpy as jnp
from jax.experimental import pallas as pl
from jax.experimental.pallas import tpu as pltpu

S = 2048
D = 1024
E = 8
DFF = 4096
CAP = 512

_INTERPRET = False


def _gating_kernel(hs_ref, wg_ref, route_ref, laux_ref, cnt_ref):
    hs = hs_ref[...]
    wg = wg_ref[...]
    logits = jnp.dot(hs, wg, preferred_element_type=jnp.float32)
    gates = jax.nn.softmax(logits, axis=-1)
    iota_e = jax.lax.broadcasted_iota(jnp.int32, (S, E), 1)
    gmax = jnp.max(gates, axis=1, keepdims=True)
    idx1 = jnp.min(jnp.where(gates == gmax, iota_e, E), axis=1, keepdims=True)
    mask1 = iota_e == idx1
    lx = jnp.where(mask1, -jnp.inf, logits)
    lmax = jnp.max(lx, axis=1, keepdims=True)
    idx2 = jnp.min(jnp.where(lx == lmax, iota_e, E), axis=1, keepdims=True)
    mask2 = iota_e == idx2
    m1f = mask1.astype(jnp.float32)
    m2f = mask2.astype(jnp.float32)
    ir = jax.lax.broadcasted_iota(jnp.int32, (S, S), 0)
    ic = jax.lax.broadcasted_iota(jnp.int32, (S, S), 1)
    ltri = (ic < ir).astype(jnp.float32)
    loc1 = jnp.dot(ltri, m1f, preferred_element_type=jnp.float32)
    count1 = jnp.sum(m1f, axis=0, keepdims=True)
    loc2 = jnp.dot(ltri, m2f, preferred_element_type=jnp.float32) + count1
    me = jnp.mean(gates, axis=0, keepdims=True)
    ce = jnp.mean(m1f, axis=0, keepdims=True)
    laux_ref[...] = jnp.sum(me * ce, axis=1, keepdims=True) * jnp.float32(E)
    cnt_ref[...] = count1.astype(jnp.int32)
    c1 = jnp.sum(jnp.where(mask1, loc1, 0.0), axis=1, keepdims=True)
    c2 = jnp.sum(jnp.where(mask2, loc2, 0.0), axis=1, keepdims=True)
    k1 = c1 < CAP
    k2 = c2 < CAP
    g1 = jnp.where(k1, jnp.sum(jnp.where(mask1, gates, 0.0), axis=1, keepdims=True), 0.0)
    g2 = jnp.where(k2, jnp.sum(jnp.where(mask2, gates, 0.0), axis=1, keepdims=True), 0.0)
    denom = jnp.maximum(g1 + g2, jnp.finfo(jnp.float32).eps)
    g1 = g1 / denom
    g2 = g2 / denom
    route = jnp.concatenate(
        [idx1.astype(jnp.float32), idx2.astype(jnp.float32), c1, c2, g1, g2,
         jnp.zeros((S, 2), jnp.float32)], axis=1)
    route_ref[...] = route


EPC = E // 2
FCH = 4
DFC = DFF // FCH


def _expert_kernel(route_ref, hsb_ref, w1_ref, b1_ref, w2_ref, b2_ref,
                   out_ref, cmb_ref, disp_ref, acc_ref):
    c = pl.program_id(0)
    ei = pl.program_id(1)
    ff = pl.program_id(2)
    e = c * EPC + ei

    @pl.when(ff == 0)
    def _():
        idx1 = route_ref[:, 0:1].astype(jnp.int32)
        idx2 = route_ref[:, 1:2].astype(jnp.int32)
        c1 = route_ref[:, 2:3].astype(jnp.int32)
        c2 = route_ref[:, 3:4].astype(jnp.int32)
        g1 = jnp.where(idx1 == e, route_ref[:, 4:5], 0.0)
        g2 = jnp.where(idx2 == e, route_ref[:, 5:6], 0.0)
        iota_c = jax.lax.broadcasted_iota(jnp.int32, (S, CAP), 1)
        cmb_f = (jnp.where(c1 == iota_c, g1, 0.0)
                 + jnp.where(c2 == iota_c, g2, 0.0))
        cmb_ref[...] = cmb_f.astype(jnp.bfloat16)
        mask = jnp.where(cmb_f > 0.0, 1.0, 0.0).astype(jnp.bfloat16)
        disp_ref[...] = jax.lax.dot_general(
            mask, hsb_ref[...], (((0,), (0,)), ((), ())),
            preferred_element_type=jnp.float32).astype(jnp.bfloat16)
        acc_ref[...] = jnp.broadcast_to(b2_ref[0, 0][None, :], (CAP, D))

    h = jnp.maximum(
        jnp.dot(disp_ref[...], w1_ref[0].astype(jnp.bfloat16),
                preferred_element_type=jnp.float32)
        + b1_ref[0, 0][None, :], 0.0).astype(jnp.bfloat16)
    acc_ref[...] += jnp.dot(h, w2_ref[0].astype(jnp.bfloat16),
                            preferred_element_type=jnp.float32)

    @pl.when(ff == FCH - 1)
    def _():
        contrib = jnp.dot(cmb_ref[...], acc_ref[...].astype(jnp.bfloat16),
                          preferred_element_type=jnp.float32)

        @pl.when(ei == 0)
        def _():
            out_ref[0] = contrib

        @pl.when(ei != 0)
        def _():
            out_ref[0] += contrib


def kernel(hidden_states, wg, w1, b1, w2, b2):
    route, laux, counts = pl.pallas_call(
        _gating_kernel,
        out_shape=(
            jax.ShapeDtypeStruct((S, 8), jnp.float32),
            jax.ShapeDtypeStruct((1, 1), jnp.float32),
            jax.ShapeDtypeStruct((1, E), jnp.int32),
        ),
        interpret=_INTERPRET,
    )(hidden_states, wg)

    hsb = hidden_states.astype(jnp.bfloat16)

    out2 = pl.pallas_call(
        _expert_kernel,
        grid=(2, EPC, FCH),
        in_specs=[
            pl.BlockSpec((S, 8), lambda c, ei, ff: (0, 0)),
            pl.BlockSpec((S, D), lambda c, ei, ff: (0, 0)),
            pl.BlockSpec((1, D, DFC), lambda c, ei, ff: (c * EPC + ei, 0, ff)),
            pl.BlockSpec((1, 1, DFC), lambda c, ei, ff: (c * EPC + ei, 0, ff)),
            pl.BlockSpec((1, DFC, D), lambda c, ei, ff: (c * EPC + ei, ff, 0)),
            pl.BlockSpec((1, 1, D), lambda c, ei, ff: (c * EPC + ei, 0, 0)),
        ],
        out_specs=pl.BlockSpec((1, S, D), lambda c, ei, ff: (c, 0, 0)),
        out_shape=jax.ShapeDtypeStruct((2, S, D), jnp.float32),
        scratch_shapes=[
            pltpu.VMEM((S, CAP), jnp.bfloat16),
            pltpu.VMEM((CAP, D), jnp.bfloat16),
            pltpu.VMEM((CAP, D), jnp.float32),
        ],
        compiler_params=pltpu.CompilerParams(
            dimension_semantics=("parallel", "arbitrary", "arbitrary"),
        ),
        interpret=_INTERPRET,
    )(route, hsb, w1, b1.reshape(E, 1, DFF), w2, b2.reshape(E, 1, D))

    return out2[0] + out2[1], laux.reshape(()), counts.reshape((E,))
